# Initial kernel scaffold; baseline (speedup 1.0000x reference)
import jax
import jax.numpy as jnp
from jax.experimental import pallas as pl


def kernel(input_ids, attention_mask, bias_table):
    raise NotImplementedError("write your pallas kernel here")


# speedup vs baseline: 118.5270x; 118.5270x over previous
import functools

import jax
import jax.numpy as jnp
import numpy as np
from jax import lax
from jax.experimental import pallas as pl

NUM_BUCKETS = 32
MAX_DISTANCE = 128
NUM_HEADS = 16
S = 2048
NSHIFT = 128
LINE_LEN = 4096
LINE_PAD = LINE_LEN + NSHIFT

ROW_BLOCK = 64


def _line_kernel(table_ref, lines_ref):
    k = lax.broadcasted_iota(jnp.int32, (NUM_HEADS, LINE_PAD), 1)
    d = k - (S - 1)
    nb = NUM_BUCKETS // 2
    rel_buckets = (d > 0).astype(jnp.int32) * nb
    ad = jnp.abs(d)
    max_exact = nb // 2
    is_small = ad < max_exact
    rp_f = jnp.maximum(ad, 1).astype(jnp.float32)
    large = max_exact + (
        jnp.log(rp_f / max_exact) / np.log(MAX_DISTANCE / max_exact) * (nb - max_exact)
    ).astype(jnp.int32)
    large = jnp.minimum(large, jnp.full_like(large, nb - 1))
    bucket = rel_buckets + jnp.where(is_small, ad, large)

    line = jnp.zeros((NUM_HEADS, LINE_PAD), jnp.float32)
    for b in range(NUM_BUCKETS):
        val = table_ref[b, :][:, None]
        line = jnp.where(bucket == b, val, line)
    for c in range(NSHIFT):
        lines_ref[c] = line[:, c : c + LINE_LEN]


def _expand_kernel(lines_ref, out_ref):
    i0 = pl.program_id(0) * ROW_BLOCK

    def body(r, _):
        s = (S - 1) - i0 - r
        c = lax.rem(s, NSHIFT)
        base = pl.multiple_of(s - c, NSHIFT)
        out_ref[:, r, :] = lines_ref[c, :, pl.ds(base, S)]
        return 0

    lax.fori_loop(0, ROW_BLOCK, body, 0)


def kernel(input_ids, attention_mask, bias_table):
    del input_ids, attention_mask
    lines = pl.pallas_call(
        _line_kernel,
        out_shape=jax.ShapeDtypeStruct((NSHIFT, NUM_HEADS, LINE_LEN), jnp.float32),
    )(bias_table)

    out = pl.pallas_call(
        _expand_kernel,
        grid=(S // ROW_BLOCK,),
        in_specs=[pl.BlockSpec((NSHIFT, NUM_HEADS, LINE_LEN), lambda g: (0, 0, 0))],
        out_specs=pl.BlockSpec((NUM_HEADS, ROW_BLOCK, S), lambda g: (0, g, 0)),
        out_shape=jax.ShapeDtypeStruct((NUM_HEADS, S, S), jnp.float32),
    )(lines)
    return out[None]


# speedup vs baseline: 140.9872x; 1.1895x over previous
import functools

import jax
import jax.numpy as jnp
import numpy as np
from jax import lax
from jax.experimental import pallas as pl
from jax.experimental.pallas import tpu as pltpu

NUM_BUCKETS = 32
MAX_DISTANCE = 128
NUM_HEADS = 16
S = 2048
NSHIFT = 128
LINE_LEN = 4096
LINE_PAD = LINE_LEN + NSHIFT

ROW_BLOCK = 64


def _compute_line():
    k = lax.broadcasted_iota(jnp.int32, (NUM_HEADS, LINE_PAD), 1)
    d = k - (S - 1)
    nb = NUM_BUCKETS // 2
    rel_buckets = (d > 0).astype(jnp.int32) * nb
    ad = jnp.abs(d)
    max_exact = nb // 2
    is_small = ad < max_exact
    rp_f = jnp.maximum(ad, 1).astype(jnp.float32)
    large = max_exact + (
        jnp.log(rp_f / max_exact) / np.log(MAX_DISTANCE / max_exact) * (nb - max_exact)
    ).astype(jnp.int32)
    large = jnp.minimum(large, jnp.full_like(large, nb - 1))
    return rel_buckets + jnp.where(is_small, ad, large)


def _fused_kernel(table_ref, out_ref, lines_ref):
    @pl.when(pl.program_id(0) == 0)
    def _build():
        bucket = _compute_line()
        line = jnp.zeros((NUM_HEADS, LINE_PAD), jnp.float32)
        for b in range(NUM_BUCKETS):
            val = table_ref[b, :][:, None]
            line = jnp.where(bucket == b, val, line)
        for c in range(NSHIFT):
            lines_ref[c] = line[:, c : c + LINE_LEN]

    i0 = pl.program_id(0) * ROW_BLOCK

    def body(r, _):
        s = (S - 1) - i0 - r
        c = lax.rem(s, NSHIFT)
        base = pl.multiple_of(s - c, NSHIFT)
        out_ref[:, r, :] = lines_ref[c, :, pl.ds(base, S)]
        return 0

    lax.fori_loop(0, ROW_BLOCK, body, 0)


def kernel(input_ids, attention_mask, bias_table):
    del input_ids, attention_mask
    out = pl.pallas_call(
        _fused_kernel,
        grid=(S // ROW_BLOCK,),
        in_specs=[pl.BlockSpec((NUM_BUCKETS, NUM_HEADS), lambda g: (0, 0))],
        out_specs=pl.BlockSpec((NUM_HEADS, ROW_BLOCK, S), lambda g: (0, g, 0)),
        out_shape=jax.ShapeDtypeStruct((NUM_HEADS, S, S), jnp.float32),
        scratch_shapes=[pltpu.VMEM((NSHIFT, NUM_HEADS, LINE_LEN), jnp.float32)],
    )(bias_table)
    return out[None]
